# baseline (speedup 1.0000x reference)
import functools

import jax
import jax.numpy as jnp
from jax import lax
from jax.experimental import pallas as pl
from jax.experimental.pallas import tpu as pltpu
from jax.experimental.pallas import tpu_sc as plsc

_LANES = 16
_UNROLL = 8
_TR, _TC = 8, 128


def _xlane_take(x, perm):
    dnums = lax.GatherDimensionNumbers(
        offset_dims=(), collapsed_slice_dims=(0,), start_index_map=(0,))
    return lax.gather(x, perm[:, None], dnums, slice_sizes=(1,),
                      mode=lax.GatherScatterMode.PROMISE_IN_BOUNDS)


def _merge(m_a, i_a, m_b, i_b):
    take_b = (m_b > m_a) | ((m_b == m_a) & (i_b < i_a))
    return jnp.where(take_b, m_b, m_a), jnp.where(take_b, i_b, i_a)


def _make_argmax_sc(rows, cols, full_rows):
    info = plsc.get_sparse_core_info()
    ncores, nsub = info.num_cores, info.num_subcores
    nworkers = ncores * nsub
    assert rows % nworkers == 0 and rows % _TR == 0 and cols % _TC == 0
    rows_per = rows // nworkers
    assert rows_per <= 3
    assert cols % (_LANES * _UNROLL) == 0
    steps = cols // (_LANES * _UNROLL)
    segs = cols // _TC
    seg_per_step = (_LANES * _UNROLL) // _TC

    mesh = plsc.VectorSubcoreMesh(core_axis_name="c", subcore_axis_name="s")

    @functools.partial(
        pl.kernel,
        out_type=jax.ShapeDtypeStruct((rows, _LANES), jnp.int32),
        mesh=mesh,
        scratch_types=[
            [pltpu.VMEM((segs, _TC), jnp.float32) for _ in range(rows_per)],
            pltpu.VMEM((rows // nworkers, _LANES), jnp.int32),
            [pltpu.SemaphoreType.DMA for _ in range(rows_per)],
        ],
    )
    def argmax_sc(x_hbm, win_hbm, bufs, cols_v, sems):
        wid = lax.axis_index("c") * nsub + lax.axis_index("s")
        row0 = wid * rows_per

        def start_in(rr, buf, sem):
            return pltpu.async_copy(
                x_hbm.at[rr // _TR, :, rr % _TR, :], buf, sem)

        copies = [start_in(row0 + r, bufs[r], sems[r])
                  for r in range(rows_per)]

        lane = lax.iota(jnp.int32, _LANES)
        neg_inf = jnp.full((_LANES,), -jnp.inf, jnp.float32)
        zero_i = jnp.zeros((_LANES,), jnp.int32)

        def row_argmax(buf):
            def step(j, carry):
                ms, tags = carry
                new_ms, new_tags = [], []
                for u in range(_UNROLL):
                    v = buf[j * seg_per_step + u // (_TC // _LANES),
                            pl.ds((u % (_TC // _LANES)) * _LANES, _LANES)]
                    gt = v > ms[u]
                    new_ms.append(jnp.where(gt, v, ms[u]))
                    new_tags.append(jnp.where(gt, j, tags[u]))
                return tuple(new_ms), tuple(new_tags)

            init = ((neg_inf,) * _UNROLL, (zero_i,) * _UNROLL)
            ms, tags = lax.fori_loop(0, steps, step, init)

            pairs = [
                (ms[u], tags[u] * (_UNROLL * _LANES) + (u * _LANES) + lane)
                for u in range(_UNROLL)
            ]
            while len(pairs) > 1:
                nxt = []
                for p in range(0, len(pairs), 2):
                    nxt.append(_merge(*pairs[p], *pairs[p + 1]))
                pairs = nxt
            m, idx = pairs[0]

            for k in (8, 4, 2, 1):
                perm = lane ^ k
                m2 = _xlane_take(m, perm)
                i2 = _xlane_take(idx, perm)
                m, idx = _merge(m, idx, m2, i2)
            return idx

        for r in range(rows_per):
            copies[r].wait()
            cols_v[r, :] = row_argmax(bufs[r])

        pltpu.sync_copy(cols_v, win_hbm.at[pl.ds(row0, rows_per)])

    return argmax_sc


def _argmax_tc_body(rows_blk, cols):
    def body(x_ref, win_ref):
        x = x_ref[...]
        idx = jnp.argmax(x, axis=1).astype(jnp.int32)[:, None]
        win_ref[...] = jnp.broadcast_to(idx, (rows_blk, _LANES))
    return body


def _make_argmax_tc(rows, cols, row_off, rows_blk=8):
    blk_off = row_off // rows_blk
    return pl.pallas_call(
        _argmax_tc_body(rows_blk, cols),
        grid=(rows // rows_blk,),
        in_specs=[pl.BlockSpec((rows_blk, cols), lambda i: (i + blk_off, 0))],
        out_specs=pl.BlockSpec((rows_blk, _LANES), lambda i: (i, 0)),
        out_shape=jax.ShapeDtypeStruct((rows, _LANES), jnp.int32),
    )


def _onehot_tc_body(rows_blk, cols, sc_blocks):
    def body(wsc_ref, wtc_ref, out_ref):
        i = pl.program_id(0)
        w = jnp.where(i < sc_blocks, wsc_ref[...], wtc_ref[...])[:, 0:1]
        col_iota = lax.broadcasted_iota(jnp.int32, (rows_blk, cols), 1)
        out_ref[...] = (col_iota == w).astype(jnp.float32)
    return body


def _make_onehot(rows, cols, sc_rows, rows_blk=32):
    sc_blocks = sc_rows // rows_blk
    return pl.pallas_call(
        _onehot_tc_body(rows_blk, cols, sc_blocks),
        grid=(rows // rows_blk,),
        in_specs=[
            pl.BlockSpec((rows_blk, _LANES), lambda i: (i % sc_blocks, 0)),
            pl.BlockSpec((rows_blk, _LANES),
                         lambda i: ((i - sc_blocks) % sc_blocks, 0)),
        ],
        out_specs=pl.BlockSpec((rows_blk, cols), lambda i: (i, 0)),
        out_shape=jax.ShapeDtypeStruct((rows, cols), jnp.float32),
    )


def kernel(tensor):
    rows, cols = tensor.shape
    split = rows // 2
    x4 = tensor.reshape(rows // _TR, _TR, cols // _TC, _TC).transpose(0, 2, 1, 3)
    win_sc = _make_argmax_sc(split, cols, rows)(x4)
    win_tc = _make_argmax_tc(split, cols, split)(tensor)
    return _make_onehot(rows, cols, split)(win_sc, win_tc)


# speedup vs baseline: 1.0057x; 1.0057x over previous
import functools

import jax
import jax.numpy as jnp
from jax import lax
from jax.experimental import pallas as pl
from jax.experimental.pallas import tpu as pltpu
from jax.experimental.pallas import tpu_sc as plsc

_LANES = 16
_UNROLL = 8
_TR, _TC = 8, 128


def _xlane_take(x, perm):
    dnums = lax.GatherDimensionNumbers(
        offset_dims=(), collapsed_slice_dims=(0,), start_index_map=(0,))
    return lax.gather(x, perm[:, None], dnums, slice_sizes=(1,),
                      mode=lax.GatherScatterMode.PROMISE_IN_BOUNDS)


def _merge(m_a, i_a, m_b, i_b):
    take_b = (m_b > m_a) | ((m_b == m_a) & (i_b < i_a))
    return jnp.where(take_b, m_b, m_a), jnp.where(take_b, i_b, i_a)


def _make_argmax_sc(rows, cols, full_rows):
    info = plsc.get_sparse_core_info()
    ncores, nsub = info.num_cores, info.num_subcores
    nworkers = ncores * nsub
    assert rows % nworkers == 0 and rows % _TR == 0 and cols % _TC == 0
    rows_per = rows // nworkers
    assert rows_per <= 3
    assert cols % (_LANES * _UNROLL) == 0
    steps = cols // (_LANES * _UNROLL)
    segs = cols // _TC
    seg_per_step = (_LANES * _UNROLL) // _TC

    mesh = plsc.VectorSubcoreMesh(core_axis_name="c", subcore_axis_name="s")

    @functools.partial(
        pl.kernel,
        out_type=jax.ShapeDtypeStruct((rows, _LANES), jnp.int32),
        mesh=mesh,
        scratch_types=[
            [pltpu.VMEM((segs, _TC), jnp.float32) for _ in range(rows_per)],
            pltpu.VMEM((rows // nworkers, _LANES), jnp.int32),
            [pltpu.SemaphoreType.DMA for _ in range(rows_per)],
        ],
    )
    def argmax_sc(x_hbm, win_hbm, bufs, cols_v, sems):
        wid = lax.axis_index("c") * nsub + lax.axis_index("s")
        row0 = wid * rows_per

        def start_in(rr, buf, sem):
            return pltpu.async_copy(
                x_hbm.at[rr // _TR, :, rr % _TR, :], buf, sem)

        copies = [start_in(row0 + r, bufs[r], sems[r])
                  for r in range(rows_per)]

        lane = lax.iota(jnp.int32, _LANES)
        neg_inf = jnp.full((_LANES,), -jnp.inf, jnp.float32)
        zero_i = jnp.zeros((_LANES,), jnp.int32)

        def row_argmax(buf):
            def step(j, carry):
                ms, tags = carry
                new_ms, new_tags = [], []
                for u in range(_UNROLL):
                    v = buf[j * seg_per_step + u // (_TC // _LANES),
                            pl.ds((u % (_TC // _LANES)) * _LANES, _LANES)]
                    gt = v > ms[u]
                    new_ms.append(jnp.where(gt, v, ms[u]))
                    new_tags.append(jnp.where(gt, j, tags[u]))
                return tuple(new_ms), tuple(new_tags)

            init = ((neg_inf,) * _UNROLL, (zero_i,) * _UNROLL)
            ms, tags = lax.fori_loop(0, steps, step, init)

            pairs = [
                (ms[u], tags[u] * (_UNROLL * _LANES) + (u * _LANES) + lane)
                for u in range(_UNROLL)
            ]
            while len(pairs) > 1:
                nxt = []
                for p in range(0, len(pairs), 2):
                    nxt.append(_merge(*pairs[p], *pairs[p + 1]))
                pairs = nxt
            m, idx = pairs[0]

            for k in (8, 4, 2, 1):
                perm = lane ^ k
                m2 = _xlane_take(m, perm)
                i2 = _xlane_take(idx, perm)
                m, idx = _merge(m, idx, m2, i2)
            return idx

        for r in range(rows_per):
            copies[r].wait()
            cols_v[r, :] = row_argmax(bufs[r])

        pltpu.sync_copy(cols_v, win_hbm.at[pl.ds(row0, rows_per)])

    return argmax_sc


def _argmax_tc_body(rows_blk, cols):
    def body(x_ref, win_ref):
        x = x_ref[...]
        col_iota = lax.broadcasted_iota(jnp.int32, (rows_blk, cols), 1)
        m = jnp.max(x, axis=1, keepdims=True)
        idx = jnp.min(jnp.where(x == m, col_iota, cols), axis=1, keepdims=True)
        win_ref[...] = jnp.broadcast_to(idx, (rows_blk, _LANES))
    return body


def _make_argmax_tc(rows, cols, row_off, rows_blk=8):
    blk_off = row_off // rows_blk
    return pl.pallas_call(
        _argmax_tc_body(rows_blk, cols),
        grid=(rows // rows_blk,),
        in_specs=[pl.BlockSpec((rows_blk, cols), lambda i: (i + blk_off, 0))],
        out_specs=pl.BlockSpec((rows_blk, _LANES), lambda i: (i, 0)),
        out_shape=jax.ShapeDtypeStruct((rows, _LANES), jnp.int32),
    )


def _onehot_tc_body(rows_blk, cols, sc_blocks):
    def body(wsc_ref, wtc_ref, out_ref):
        i = pl.program_id(0)
        w = jnp.where(i < sc_blocks, wsc_ref[...], wtc_ref[...])[:, 0:1]
        col_iota = lax.broadcasted_iota(jnp.int32, (rows_blk, cols), 1)
        out_ref[...] = (col_iota == w).astype(jnp.float32)
    return body


def _make_onehot(rows, cols, sc_rows, rows_blk=32):
    sc_blocks = sc_rows // rows_blk
    return pl.pallas_call(
        _onehot_tc_body(rows_blk, cols, sc_blocks),
        grid=(rows // rows_blk,),
        in_specs=[
            pl.BlockSpec((rows_blk, _LANES), lambda i: (i % sc_blocks, 0)),
            pl.BlockSpec((rows_blk, _LANES),
                         lambda i: ((i - sc_blocks) % sc_blocks, 0)),
        ],
        out_specs=pl.BlockSpec((rows_blk, cols), lambda i: (i, 0)),
        out_shape=jax.ShapeDtypeStruct((rows, cols), jnp.float32),
    )


def kernel(tensor):
    rows, cols = tensor.shape
    split = rows // 2
    x4 = tensor.reshape(rows // _TR, _TR, cols // _TC, _TC).transpose(0, 2, 1, 3)
    win_sc = _make_argmax_sc(split, cols, rows)(x4)
    win_tc = _make_argmax_tc(split, cols, split)(tensor)
    return _make_onehot(rows, cols, split)(win_sc, win_tc)
